# baseline (device time: 18217 ns/iter reference)
import jax
import jax.numpy as jnp
from jax import lax
from jax.experimental import pallas as pl
from jax.experimental.pallas import tpu as pltpu

N_DEV = 4
B = 2
SQ = 256
SKV = 256
HQ = 4
DH = 64
D = 512
HD = HQ * DH
HALO = 128
WIN = 128
KW = SKV + 2 * HALO


def _body(x_hbm, wq_hbm, kt_hbm, vt_hbm, wo_hbm, out_hbm,
          x_vmem, wq_vmem, kt_f32, vt_f32, wo_vmem, out_vmem,
          kt_bf, vt_bf, kleft, kright, vleft, vright,
          send_sems, recv_sems, in_sems, out_sems):
    bf16 = jnp.bfloat16
    pos = lax.axis_index("i")
    left = (pos - 1) % N_DEV
    right = (pos + 1) % N_DEV

    fetches = [
        pltpu.make_async_copy(src, dst, in_sems.at[i])
        for i, (src, dst) in enumerate((
            (kt_hbm, kt_f32), (vt_hbm, vt_f32),
            (x_hbm, x_vmem), (wq_hbm, wq_vmem), (wo_hbm, wo_vmem),
        ))
    ]
    for f in fetches:
        f.start()

    barrier_sem = pltpu.get_barrier_semaphore()
    for nbr in (left, right):
        pl.semaphore_signal(
            barrier_sem, inc=1,
            device_id=(nbr,), device_id_type=pl.DeviceIdType.MESH,
        )
    pl.semaphore_wait(barrier_sem, 2)

    fetches[0].wait()
    fetches[1].wait()
    for b in range(B):
        for half in range(2):
            cols = slice(half * HALO, (half + 1) * HALO)
            kt_bf[2 * b + half] = (
                kt_f32[b, :, :, cols].reshape(HD, HALO).astype(bf16))
            vt_bf[2 * b + half] = (
                vt_f32[b, :, :, cols].reshape(HD, HALO).astype(bf16))

    transfers = []
    for b in range(B):
        transfers += [
            (kt_bf.at[2 * b + 1], kleft.at[b], right),
            (vt_bf.at[2 * b + 1], vleft.at[b], right),
            (kt_bf.at[2 * b], kright.at[b], left),
            (vt_bf.at[2 * b], vright.at[b], left),
        ]
    rdmas = []
    for idx, (src, dst, tgt) in enumerate(transfers):
        rdma = pltpu.make_async_remote_copy(
            src_ref=src, dst_ref=dst,
            send_sem=send_sems.at[idx], recv_sem=recv_sems.at[idx],
            device_id=(tgt,), device_id_type=pl.DeviceIdType.MESH,
        )
        rdma.start()
        rdmas.append(rdma)

    fetches[2].wait()
    fetches[3].wait()
    wq = wq_vmem[...].astype(bf16)
    q = []
    for b in range(B):
        qb = lax.dot_general(
            x_vmem[b].astype(bf16), wq,
            (((1,), (0,)), ((), ())), preferred_element_type=jnp.float32,
        )
        q.append((qb * 0.125).astype(bf16))

    QB, KB = SQ // 2, SQ // 2 + 2 * HALO
    i2 = lax.broadcasted_iota(jnp.int32, (QB, KB), 0)
    j2 = lax.broadcasted_iota(jnp.int32, (QB, KB), 1)
    window = jnp.abs(i2 + HALO - j2) <= WIN
    biases = []
    for t in range(2):
        ki = pos * SKV - HALO + t * QB + j2
        mask = window & (ki >= 0) & (ki < N_DEV * SKV)
        biases.append(jnp.where(mask, 0.0, -1e9).astype(jnp.float32))

    fetches[4].wait()
    wo = wo_vmem[...].astype(bf16)

    out_copies = []
    for b in range(B):
        for rdma in rdmas[4 * b:4 * b + 4]:
            rdma.wait_recv()
        ctx_heads = []
        for h in range(HQ):
            rows = slice(h * DH, (h + 1) * DH)
            loc0_k, loc1_k = kt_bf[2 * b][rows], kt_bf[2 * b + 1][rows]
            loc0_v, loc1_v = vt_bf[2 * b][rows], vt_bf[2 * b + 1][rows]
            kts = [
                jnp.concatenate([kleft[b][rows], loc0_k, loc1_k], axis=1),
                jnp.concatenate([loc0_k, loc1_k, kright[b][rows]], axis=1),
            ]
            vts = [
                jnp.concatenate([vleft[b][rows], loc0_v, loc1_v], axis=1),
                jnp.concatenate([loc0_v, loc1_v, vright[b][rows]], axis=1),
            ]
            ctx_blocks = []
            for t in range(2):
                s = lax.dot_general(
                    q[b][t * QB:(t + 1) * QB, rows], kts[t],
                    (((1,), (0,)), ((), ())),
                    preferred_element_type=jnp.float32,
                ) + biases[t]
                e = jnp.exp(s)
                w = e * (1.0 / jnp.sum(e, axis=1, keepdims=True))
                ctx_blocks.append(lax.dot_general(
                    w.astype(bf16), vts[t], (((1,), (1,)), ((), ())),
                    preferred_element_type=jnp.float32,
                ))
            ctx_heads.append(jnp.concatenate(ctx_blocks, axis=0))
        ctx_b = jnp.concatenate(ctx_heads, axis=1).astype(bf16)
        out_vmem[b] = lax.dot_general(
            ctx_b, wo,
            (((1,), (0,)), ((), ())), preferred_element_type=jnp.float32,
        ).astype(bf16)
        oc = pltpu.make_async_copy(
            out_vmem.at[b], out_hbm.at[b], out_sems.at[b])
        oc.start()
        out_copies.append(oc)

    for rdma in rdmas:
        rdma.wait_send()
    for oc in out_copies:
        oc.wait()


def kernel(x, Wq, K_ext, V_ext, Wo):
    kt = jnp.transpose(K_ext, (0, 2, 3, 1))
    vt = jnp.transpose(V_ext, (0, 2, 3, 1))
    halo = pltpu.VMEM((B, HD, HALO), jnp.bfloat16)
    return pl.pallas_call(
        _body,
        out_shape=jax.ShapeDtypeStruct((B, SQ, D), jnp.bfloat16),
        in_specs=[pl.BlockSpec(memory_space=pl.ANY)] * 5,
        out_specs=pl.BlockSpec(memory_space=pl.ANY),
        scratch_shapes=[
            pltpu.VMEM((B, SQ, D), jnp.float32),
            pltpu.VMEM((D, HD), jnp.float32),
            pltpu.VMEM((B, HQ, DH, SKV), jnp.float32),
            pltpu.VMEM((B, HQ, DH, SKV), jnp.float32),
            pltpu.VMEM((HD, D), jnp.float32),
            pltpu.VMEM((B, SQ, D), jnp.bfloat16),
            pltpu.VMEM((B * 2, HD, HALO), jnp.bfloat16),
            pltpu.VMEM((B * 2, HD, HALO), jnp.bfloat16),
            halo, halo, halo, halo,
            pltpu.SemaphoreType.DMA((8,)),
            pltpu.SemaphoreType.DMA((8,)),
            pltpu.SemaphoreType.DMA((5,)),
            pltpu.SemaphoreType.DMA((B,)),
        ],
        compiler_params=pltpu.CompilerParams(collective_id=0),
    )(x, Wq, kt, vt, Wo)


# device time: 15859 ns/iter; 1.1487x vs baseline; 1.1487x over previous
import jax
import jax.numpy as jnp
from jax import lax
from jax.experimental import pallas as pl
from jax.experimental.pallas import tpu as pltpu

N_DEV = 4
B = 2
SQ = 256
SKV = 256
HQ = 4
DH = 64
D = 512
HD = HQ * DH
HALO = 128
WIN = 128
KW = SKV + 2 * HALO


def _body(x_hbm, wq_hbm, kt_hbm, vt_hbm, wo_hbm, out_hbm,
          x_vmem, wq_vmem, kt_f32, vt_f32, wo_vmem, out_vmem,
          kt_bf, vt_bf, kleft, kright, vleft, vright,
          send_sems, recv_sems, in_sems, out_sems):
    bf16 = jnp.bfloat16
    pos = lax.axis_index("i")
    left = (pos - 1) % N_DEV
    right = (pos + 1) % N_DEV

    fetches = [
        pltpu.make_async_copy(src, dst, in_sems.at[i])
        for i, (src, dst) in enumerate((
            (kt_hbm, kt_f32), (vt_hbm, vt_f32),
            (x_hbm, x_vmem), (wq_hbm, wq_vmem), (wo_hbm, wo_vmem),
        ))
    ]
    for f in fetches:
        f.start()

    barrier_sem = pltpu.get_barrier_semaphore()
    for nbr in (left, right):
        pl.semaphore_signal(
            barrier_sem, inc=1,
            device_id=(nbr,), device_id_type=pl.DeviceIdType.MESH,
        )
    pl.semaphore_wait(barrier_sem, 2)

    fetches[0].wait()
    fetches[1].wait()
    for b in range(B):
        for half in range(2):
            cols = slice(half * HALO, (half + 1) * HALO)
            kt_bf[2 * b + half] = (
                kt_f32[b, :, :, cols].reshape(HD, HALO).astype(bf16))
            vt_bf[2 * b + half] = (
                vt_f32[b, :, :, cols].reshape(HD, HALO).astype(bf16))

    transfers = []
    for b in range(B):
        transfers += [
            (kt_bf.at[2 * b + 1], kleft.at[b], right),
            (vt_bf.at[2 * b + 1], vleft.at[b], right),
            (kt_bf.at[2 * b], kright.at[b], left),
            (vt_bf.at[2 * b], vright.at[b], left),
        ]
    rdmas = []
    for idx, (src, dst, tgt) in enumerate(transfers):
        rdma = pltpu.make_async_remote_copy(
            src_ref=src, dst_ref=dst,
            send_sem=send_sems.at[idx], recv_sem=recv_sems.at[idx],
            device_id=(tgt,), device_id_type=pl.DeviceIdType.MESH,
        )
        rdma.start()
        rdmas.append(rdma)

    fetches[2].wait()
    fetches[3].wait()
    wq = wq_vmem[...].astype(bf16)
    q = []
    for b in range(B):
        qb = lax.dot_general(
            x_vmem[b].astype(bf16), wq,
            (((1,), (0,)), ((), ())), preferred_element_type=jnp.float32,
        )
        q.append((qb * 0.125).astype(bf16))

    i2 = lax.broadcasted_iota(jnp.int32, (SQ, KW), 0)
    j2 = lax.broadcasted_iota(jnp.int32, (SQ, KW), 1)
    ki = pos * SKV - HALO + j2
    mask = (jnp.abs(i2 + HALO - j2) <= WIN) & (ki >= 0) & (ki < N_DEV * SKV)
    bias = jnp.where(mask, 0.0, -1e9).astype(jnp.float32)

    fetches[4].wait()
    wo = wo_vmem[...].astype(bf16)

    out_copies = []
    for b in range(B):
        for rdma in rdmas[4 * b:4 * b + 4]:
            rdma.wait_recv()
        ctx_heads = []
        for h in range(HQ):
            rows = slice(h * DH, (h + 1) * DH)
            khT = jnp.concatenate(
                [kleft[b][rows], kt_bf[2 * b][rows],
                 kt_bf[2 * b + 1][rows], kright[b][rows]], axis=1)
            vhT = jnp.concatenate(
                [vleft[b][rows], vt_bf[2 * b][rows],
                 vt_bf[2 * b + 1][rows], vright[b][rows]], axis=1)
            s = lax.dot_general(
                q[b][:, rows], khT, (((1,), (0,)), ((), ())),
                preferred_element_type=jnp.float32,
            ) + bias
            e = jnp.exp(s)
            w = e * (1.0 / jnp.sum(e, axis=1, keepdims=True))
            ctx = lax.dot_general(
                w.astype(bf16), vhT, (((1,), (1,)), ((), ())),
                preferred_element_type=jnp.float32,
            )
            ctx_heads.append(ctx)
        ctx_b = jnp.concatenate(ctx_heads, axis=1).astype(bf16)
        out_vmem[b] = lax.dot_general(
            ctx_b, wo,
            (((1,), (0,)), ((), ())), preferred_element_type=jnp.float32,
        ).astype(bf16)
        oc = pltpu.make_async_copy(
            out_vmem.at[b], out_hbm.at[b], out_sems.at[b])
        oc.start()
        out_copies.append(oc)

    for rdma in rdmas:
        rdma.wait_send()
    for oc in out_copies:
        oc.wait()


def kernel(x, Wq, K_ext, V_ext, Wo):
    kt = jnp.transpose(K_ext, (0, 2, 3, 1))
    vt = jnp.transpose(V_ext, (0, 2, 3, 1))
    halo = pltpu.VMEM((B, HD, HALO), jnp.bfloat16)
    return pl.pallas_call(
        _body,
        out_shape=jax.ShapeDtypeStruct((B, SQ, D), jnp.bfloat16),
        in_specs=[pl.BlockSpec(memory_space=pl.ANY)] * 5,
        out_specs=pl.BlockSpec(memory_space=pl.ANY),
        scratch_shapes=[
            pltpu.VMEM((B, SQ, D), jnp.float32),
            pltpu.VMEM((D, HD), jnp.float32),
            pltpu.VMEM((B, HQ, DH, SKV), jnp.float32),
            pltpu.VMEM((B, HQ, DH, SKV), jnp.float32),
            pltpu.VMEM((HD, D), jnp.float32),
            pltpu.VMEM((B, SQ, D), jnp.bfloat16),
            pltpu.VMEM((B * 2, HD, HALO), jnp.bfloat16),
            pltpu.VMEM((B * 2, HD, HALO), jnp.bfloat16),
            halo, halo, halo, halo,
            pltpu.SemaphoreType.DMA((8,)),
            pltpu.SemaphoreType.DMA((8,)),
            pltpu.SemaphoreType.DMA((5,)),
            pltpu.SemaphoreType.DMA((B,)),
        ],
        compiler_params=pltpu.CompilerParams(collective_id=0),
    )(x, Wq, kt, vt, Wo)


# device time: 10260 ns/iter; 1.7755x vs baseline; 1.5457x over previous
import jax
import jax.numpy as jnp
from jax import lax
from jax.experimental import pallas as pl
from jax.experimental.pallas import tpu as pltpu

N_DEV = 4
B = 2
SQ = 256
SKV = 256
HQ = 4
DH = 64
D = 512
HD = HQ * DH
HALO = 128
WIN = 128
KW = SKV + 2 * HALO


def _body(x_hbm, wq_hbm, kt_hbm, vt_hbm, wo_hbm, out_hbm,
          x_vmem, wq_vmem, kt_f32, vt_f32, wo_vmem, out_vmem,
          kt_bf, vt_bf, kleft, kright, vleft, vright,
          send_sems, recv_sems, in_sems, out_sems):
    bf16 = jnp.bfloat16
    pos = lax.axis_index("i")
    left = (pos - 1) % N_DEV
    right = (pos + 1) % N_DEV

    fetches = [
        pltpu.make_async_copy(src, dst, in_sems.at[i])
        for i, (src, dst) in enumerate((
            (kt_hbm, kt_f32), (vt_hbm, vt_f32),
            (x_hbm, x_vmem), (wq_hbm, wq_vmem), (wo_hbm, wo_vmem),
        ))
    ]
    for f in fetches:
        f.start()

    barrier_sem = pltpu.get_barrier_semaphore()
    for nbr in (left, right):
        pl.semaphore_signal(
            barrier_sem, inc=1,
            device_id=(nbr,), device_id_type=pl.DeviceIdType.MESH,
        )
    pl.semaphore_wait(barrier_sem, 2)

    fetches[0].wait()
    fetches[1].wait()
    for b in range(B):
        for half in range(2):
            cols = slice(half * HALO, (half + 1) * HALO)
            kt_bf[2 * b + half] = (
                kt_f32[b, :, :, cols].reshape(HD, HALO).astype(bf16))
            vt_bf[2 * b + half] = (
                vt_f32[b, :, :, cols].reshape(HD, HALO).astype(bf16))

    transfers = []
    for b in range(B):
        transfers += [
            (kt_bf.at[2 * b + 1], kleft.at[b], right),
            (vt_bf.at[2 * b + 1], vleft.at[b], right),
            (kt_bf.at[2 * b], kright.at[b], left),
            (vt_bf.at[2 * b], vright.at[b], left),
        ]
    rdmas = []
    for idx, (src, dst, tgt) in enumerate(transfers):
        rdma = pltpu.make_async_remote_copy(
            src_ref=src, dst_ref=dst,
            send_sem=send_sems.at[idx], recv_sem=recv_sems.at[idx],
            device_id=(tgt,), device_id_type=pl.DeviceIdType.MESH,
        )
        rdma.start()
        rdmas.append(rdma)

    fetches[2].wait()
    fetches[3].wait()
    wq = wq_vmem[...].astype(bf16)
    q = []
    for b in range(B):
        qb = lax.dot_general(
            x_vmem[b].astype(bf16), wq,
            (((1,), (0,)), ((), ())), preferred_element_type=jnp.float32,
        )
        q.append((qb * 0.125).astype(bf16))

    i2 = lax.broadcasted_iota(jnp.int32, (SQ, KW), 0)
    j2 = lax.broadcasted_iota(jnp.int32, (SQ, KW), 1)
    ki = pos * SKV - HALO + j2
    mask = (jnp.abs(i2 + HALO - j2) <= WIN) & (ki >= 0) & (ki < N_DEV * SKV)
    bias = jnp.where(mask, 0.0, -1e9).astype(jnp.float32)

    fetches[4].wait()
    wo = wo_vmem[...].astype(bf16)

    out_copies = []
    for b in range(B):
        for rdma in rdmas[4 * b:4 * b + 4]:
            rdma.wait_recv()
        ctx_heads = []
        for h in range(HQ):
            rows = slice(h * DH, (h + 1) * DH)
            khT = jnp.concatenate(
                [kleft[b][rows], kt_bf[2 * b][rows],
                 kt_bf[2 * b + 1][rows], kright[b][rows]], axis=1)
            vhT = jnp.concatenate(
                [vleft[b][rows], vt_bf[2 * b][rows],
                 vt_bf[2 * b + 1][rows], vright[b][rows]], axis=1)
            s = lax.dot_general(
                q[b][:, rows], khT, (((1,), (0,)), ((), ())),
                preferred_element_type=jnp.float32,
            ) + bias
            e = jnp.exp(s)
            w = e * (1.0 / jnp.sum(e, axis=1, keepdims=True))
            ctx = lax.dot_general(
                w.astype(bf16), vhT, (((1,), (1,)), ((), ())),
                preferred_element_type=jnp.float32,
            )
            ctx_heads.append(ctx)
        ctx_b = jnp.concatenate(ctx_heads, axis=1).astype(bf16)
        out_vmem[b] = lax.dot_general(
            ctx_b, wo,
            (((1,), (0,)), ((), ())), preferred_element_type=jnp.float32,
        ).astype(bf16)
        oc = pltpu.make_async_copy(
            out_vmem.at[b], out_hbm.at[b], out_sems.at[b])
        oc.start()
        out_copies.append(oc)

    for rdma in rdmas:
        rdma.wait_send()
    for oc in out_copies:
        oc.wait()


def kernel(x, Wq, K_ext, V_ext, Wo):
    kt = jnp.transpose(K_ext, (0, 2, 3, 1))
    vt = jnp.transpose(V_ext, (0, 2, 3, 1))
    hbm = pltpu.MemorySpace.HBM
    args = [pltpu.with_memory_space_constraint(a, hbm)
            for a in (x, Wq, kt, vt, Wo)]
    halo = pltpu.VMEM((B, HD, HALO), jnp.bfloat16)
    return pl.pallas_call(
        _body,
        out_shape=jax.ShapeDtypeStruct((B, SQ, D), jnp.bfloat16),
        in_specs=[pl.BlockSpec(memory_space=hbm)] * 5,
        out_specs=pl.BlockSpec(memory_space=hbm),
        scratch_shapes=[
            pltpu.VMEM((B, SQ, D), jnp.float32),
            pltpu.VMEM((D, HD), jnp.float32),
            pltpu.VMEM((B, HQ, DH, SKV), jnp.float32),
            pltpu.VMEM((B, HQ, DH, SKV), jnp.float32),
            pltpu.VMEM((HD, D), jnp.float32),
            pltpu.VMEM((B, SQ, D), jnp.bfloat16),
            pltpu.VMEM((B * 2, HD, HALO), jnp.bfloat16),
            pltpu.VMEM((B * 2, HD, HALO), jnp.bfloat16),
            halo, halo, halo, halo,
            pltpu.SemaphoreType.DMA((8,)),
            pltpu.SemaphoreType.DMA((8,)),
            pltpu.SemaphoreType.DMA((5,)),
            pltpu.SemaphoreType.DMA((B,)),
        ],
        compiler_params=pltpu.CompilerParams(collective_id=0),
    )(*args)


# device time: 9560 ns/iter; 1.9055x vs baseline; 1.0732x over previous
import jax
import jax.numpy as jnp
from jax import lax
from jax.experimental import pallas as pl
from jax.experimental.pallas import tpu as pltpu

N_DEV = 4
B = 2
SQ = 256
SKV = 256
HQ = 4
DH = 64
D = 512
HD = HQ * DH
HALO = 128
WIN = 128
KW = SKV + 2 * HALO


def _body(x_hbm, wq_hbm, kt_hbm, vt_hbm, wo_hbm, out_hbm,
          x_vmem, wq_vmem, kt_f32, vt_f32, wo_vmem, out_vmem,
          kt_bf, vt_bf, kleft, kright, vleft, vright,
          send_sems, recv_sems, in_sems, out_sems):
    bf16 = jnp.bfloat16
    pos = lax.axis_index("i")
    left = (pos - 1) % N_DEV
    right = (pos + 1) % N_DEV

    fetches = [
        pltpu.make_async_copy(src, dst, in_sems.at[i])
        for i, (src, dst) in enumerate((
            (kt_hbm, kt_f32), (vt_hbm, vt_f32),
            (x_hbm, x_vmem), (wq_hbm, wq_vmem), (wo_hbm, wo_vmem),
        ))
    ]
    for f in fetches:
        f.start()

    barrier_sem = pltpu.get_barrier_semaphore()
    for nbr in (left, right):
        pl.semaphore_signal(
            barrier_sem, inc=1,
            device_id=(nbr,), device_id_type=pl.DeviceIdType.MESH,
        )
    pl.semaphore_wait(barrier_sem, 2)

    fetches[0].wait()
    fetches[1].wait()
    k_rdmas, v_rdmas = [], []
    sem_idx = 0
    for b in range(B):
        for half in range(2):
            cols = slice(half * HALO, (half + 1) * HALO)
            kt_bf[2 * b + half] = (
                kt_f32[b, :, :, cols].reshape(HD, HALO).astype(bf16))
        for src, dst, tgt in (
            (kt_bf.at[2 * b + 1], kleft.at[b], right),
            (kt_bf.at[2 * b], kright.at[b], left),
        ):
            rdma = pltpu.make_async_remote_copy(
                src_ref=src, dst_ref=dst,
                send_sem=send_sems.at[sem_idx],
                recv_sem=recv_sems.at[sem_idx],
                device_id=(tgt,), device_id_type=pl.DeviceIdType.MESH,
            )
            rdma.start()
            k_rdmas.append(rdma)
            sem_idx += 1
        for half in range(2):
            cols = slice(half * HALO, (half + 1) * HALO)
            vt_bf[2 * b + half] = (
                vt_f32[b, :, :, cols].reshape(HD, HALO).astype(bf16))
        for src, dst, tgt in (
            (vt_bf.at[2 * b + 1], vleft.at[b], right),
            (vt_bf.at[2 * b], vright.at[b], left),
        ):
            rdma = pltpu.make_async_remote_copy(
                src_ref=src, dst_ref=dst,
                send_sem=send_sems.at[sem_idx],
                recv_sem=recv_sems.at[sem_idx],
                device_id=(tgt,), device_id_type=pl.DeviceIdType.MESH,
            )
            rdma.start()
            v_rdmas.append(rdma)
            sem_idx += 1
    rdmas = k_rdmas + v_rdmas

    fetches[2].wait()
    fetches[3].wait()
    wq = wq_vmem[...].astype(bf16)
    q = []
    for b in range(B):
        qb = lax.dot_general(
            x_vmem[b].astype(bf16), wq,
            (((1,), (0,)), ((), ())), preferred_element_type=jnp.float32,
        )
        q.append((qb * 0.125).astype(bf16))

    i2 = lax.broadcasted_iota(jnp.int32, (SQ, KW), 0)
    j2 = lax.broadcasted_iota(jnp.int32, (SQ, KW), 1)
    ki = pos * SKV - HALO + j2
    mask = (jnp.abs(i2 + HALO - j2) <= WIN) & (ki >= 0) & (ki < N_DEV * SKV)
    bias = jnp.where(mask, 0.0, -1e9).astype(jnp.float32)

    fetches[4].wait()
    wo = wo_vmem[...].astype(bf16)

    out_copies = []
    for b in range(B):
        for rdma in k_rdmas[2 * b:2 * b + 2]:
            rdma.wait_recv()
        ws = []
        for h in range(HQ):
            rows = slice(h * DH, (h + 1) * DH)
            khT = jnp.concatenate(
                [kleft[b][rows], kt_bf[2 * b][rows],
                 kt_bf[2 * b + 1][rows], kright[b][rows]], axis=1)
            s = lax.dot_general(
                q[b][:, rows], khT, (((1,), (0,)), ((), ())),
                preferred_element_type=jnp.float32,
            ) + bias
            e = jnp.exp(s)
            ws.append(
                (e * (1.0 / jnp.sum(e, axis=1, keepdims=True))).astype(bf16))
        for rdma in v_rdmas[2 * b:2 * b + 2]:
            rdma.wait_recv()
        ctx_heads = []
        for h in range(HQ):
            rows = slice(h * DH, (h + 1) * DH)
            vhT = jnp.concatenate(
                [vleft[b][rows], vt_bf[2 * b][rows],
                 vt_bf[2 * b + 1][rows], vright[b][rows]], axis=1)
            ctx_heads.append(lax.dot_general(
                ws[h], vhT, (((1,), (1,)), ((), ())),
                preferred_element_type=jnp.float32,
            ))
        ctx_b = jnp.concatenate(ctx_heads, axis=1).astype(bf16)
        out_vmem[b] = lax.dot_general(
            ctx_b, wo,
            (((1,), (0,)), ((), ())), preferred_element_type=jnp.float32,
        ).astype(bf16)
        oc = pltpu.make_async_copy(
            out_vmem.at[b], out_hbm.at[b], out_sems.at[b])
        oc.start()
        out_copies.append(oc)

    for rdma in rdmas:
        rdma.wait_send()
    for oc in out_copies:
        oc.wait()


def kernel(x, Wq, K_ext, V_ext, Wo):
    kt = jnp.transpose(K_ext, (0, 2, 3, 1))
    vt = jnp.transpose(V_ext, (0, 2, 3, 1))
    hbm = pltpu.MemorySpace.HBM
    args = [pltpu.with_memory_space_constraint(a, hbm)
            for a in (x, Wq, kt, vt, Wo)]
    halo = pltpu.VMEM((B, HD, HALO), jnp.bfloat16)
    return pl.pallas_call(
        _body,
        out_shape=jax.ShapeDtypeStruct((B, SQ, D), jnp.bfloat16),
        in_specs=[pl.BlockSpec(memory_space=hbm)] * 5,
        out_specs=pl.BlockSpec(memory_space=hbm),
        scratch_shapes=[
            pltpu.VMEM((B, SQ, D), jnp.float32),
            pltpu.VMEM((D, HD), jnp.float32),
            pltpu.VMEM((B, HQ, DH, SKV), jnp.float32),
            pltpu.VMEM((B, HQ, DH, SKV), jnp.float32),
            pltpu.VMEM((HD, D), jnp.float32),
            pltpu.VMEM((B, SQ, D), jnp.bfloat16),
            pltpu.VMEM((B * 2, HD, HALO), jnp.bfloat16),
            pltpu.VMEM((B * 2, HD, HALO), jnp.bfloat16),
            halo, halo, halo, halo,
            pltpu.SemaphoreType.DMA((8,)),
            pltpu.SemaphoreType.DMA((8,)),
            pltpu.SemaphoreType.DMA((5,)),
            pltpu.SemaphoreType.DMA((B,)),
        ],
        compiler_params=pltpu.CompilerParams(collective_id=0),
    )(*args)
